# baseline (speedup 1.0000x reference)
import functools

import jax
import jax.numpy as jnp
from jax import lax
from jax.experimental import pallas as pl
from jax.experimental.pallas import tpu as pltpu
from jax.experimental.pallas import tpu_sc as plsc

FIELD = 26
BATCH = 16384
LANES = 16
NC = 2
NS = 16
NW = NC * NS
ROWS_W = BATCH // NW
CHUNKS = ROWS_W // LANES
FLAT = FIELD * ROWS_W


def _sc_body(ids_hbm, vals_hbm, w_hbm, b_hbm, out_hbm,
             idx_v, vals_v, g_v, out_v, b_v, sem):
    c = lax.axis_index("c")
    s = lax.axis_index("s")
    wid = s * NC + c
    base = wid * FLAT

    pltpu.sync_copy(ids_hbm.at[pl.ds(base, FLAT)], idx_v)
    gat = pltpu.async_copy(w_hbm.at[idx_v], g_v, sem)
    pltpu.sync_copy(vals_hbm.at[pl.ds(base, FLAT)], vals_v)
    pltpu.sync_copy(b_hbm, b_v)
    gat.wait()

    bias = b_v[...]
    lane_off = lax.iota(jnp.int32, LANES) * FIELD

    def chunk(ci, fidx0):
        acc = jnp.zeros((LANES,), jnp.float32)
        fidx = fidx0
        for j in range(FIELD):
            acc = acc + plsc.load_gather(g_v, [fidx]) * plsc.load_gather(vals_v, [fidx])
            fidx = fidx + 1
        z = acc + bias
        out_v[pl.ds(ci * LANES, LANES)] = 1.0 / (1.0 + jnp.exp(-z))
        return fidx0 + (LANES * FIELD)

    lax.fori_loop(0, CHUNKS, chunk, lane_off)
    pltpu.sync_copy(out_v, out_hbm.at[pl.ds(wid * ROWS_W, ROWS_W)])


_sc_kernel = functools.partial(
    pl.kernel,
    out_type=jax.ShapeDtypeStruct((BATCH,), jnp.float32),
    mesh=plsc.VectorSubcoreMesh(core_axis_name="c", subcore_axis_name="s"),
    compiler_params=pltpu.CompilerParams(needs_layout_passes=False),
    scratch_types=[
        pltpu.VMEM((FLAT,), jnp.int32),
        pltpu.VMEM((FLAT,), jnp.float32),
        pltpu.VMEM((FLAT,), jnp.float32),
        pltpu.VMEM((ROWS_W,), jnp.float32),
        pltpu.VMEM((LANES,), jnp.float32),
        pltpu.SemaphoreType.DMA,
    ],
)(_sc_body)


def kernel(feat_ids, feat_vals, LR_W, LR_B):
    b16 = jnp.broadcast_to(LR_B, (LANES,))
    return _sc_kernel(feat_ids.reshape(-1), feat_vals.reshape(-1), LR_W, b16)


# speedup vs baseline: 1.0647x; 1.0647x over previous
import functools

import jax
import jax.numpy as jnp
from jax import lax
from jax.experimental import pallas as pl
from jax.experimental.pallas import tpu as pltpu
from jax.experimental.pallas import tpu_sc as plsc

FIELD = 26
BATCH = 16384
LANES = 16
NC = 2
NS = 16
NW = NC * NS
ROWS_W = BATCH // NW
FLAT = FIELD * ROWS_W
NCHUNK = 8
ROWS_C = ROWS_W // NCHUNK
SPAN = FIELD * ROWS_C
PACKED = 2 * FLAT + 128


def _sc_body(packed_hbm, w_hbm, out_hbm, buf_v, g_v, out_v, sems):
    c = lax.axis_index("c")
    s = lax.axis_index("s")
    wid = s * NC + c

    pltpu.sync_copy(packed_hbm.at[wid, pl.ds(0, FLAT)], buf_v.at[pl.ds(0, FLAT)])
    for ck in range(NCHUNK):
        pltpu.async_copy(
            w_hbm.at[buf_v.at[pl.ds(ck * SPAN, SPAN)]],
            g_v.at[pl.ds(ck * SPAN, SPAN)],
            sems.at[ck])
    pltpu.sync_copy(packed_hbm.at[wid, pl.ds(FLAT, FLAT + 128)],
                    buf_v.at[pl.ds(FLAT, FLAT + 128)])

    bias = plsc.bitcast(buf_v[pl.ds(2 * FLAT, LANES)], jnp.float32)
    lane_off = lax.iota(jnp.int32, LANES) * FIELD

    def chunk(ck, fidx0):
        pltpu.make_async_copy(
            w_hbm.at[buf_v.at[pl.ds(ck * SPAN, SPAN)]],
            g_v.at[pl.ds(ck * SPAN, SPAN)],
            sems.at[ck]).wait()
        fx = fidx0
        for t in range(ROWS_C // LANES):
            acc = jnp.zeros((LANES,), jnp.float32)
            fidx = fx
            for j in range(FIELD):
                w = plsc.load_gather(g_v, [fidx])
                v = plsc.bitcast(plsc.load_gather(buf_v, [fidx + FLAT]), jnp.float32)
                acc = acc + w * v
                fidx = fidx + 1
            z = acc + bias
            out_v[pl.ds(ck * ROWS_C + t * LANES, LANES)] = 1.0 / (1.0 + jnp.exp(-z))
            fx = fx + (LANES * FIELD)
        return fx

    lax.fori_loop(0, NCHUNK, chunk, lane_off)
    pltpu.sync_copy(out_v, out_hbm.at[pl.ds(wid * ROWS_W, ROWS_W)])


_sc_kernel = functools.partial(
    pl.kernel,
    out_type=jax.ShapeDtypeStruct((BATCH,), jnp.float32),
    mesh=plsc.VectorSubcoreMesh(core_axis_name="c", subcore_axis_name="s"),
    compiler_params=pltpu.CompilerParams(needs_layout_passes=False),
    scratch_types=[
        pltpu.VMEM((PACKED,), jnp.int32),
        pltpu.VMEM((FLAT,), jnp.float32),
        pltpu.VMEM((ROWS_W,), jnp.float32),
        pltpu.SemaphoreType.DMA((NCHUNK,)),
    ],
)(_sc_body)


def kernel(feat_ids, feat_vals, LR_W, LR_B):
    ids_p = feat_ids.reshape(NW, FLAT)
    vals_p = lax.bitcast_convert_type(feat_vals, jnp.int32).reshape(NW, FLAT)
    bias_p = jnp.broadcast_to(lax.bitcast_convert_type(LR_B, jnp.int32), (NW, 128))
    packed = jnp.concatenate([ids_p, vals_p, bias_p], axis=1)
    return _sc_kernel(packed, LR_W)


# speedup vs baseline: 1.4643x; 1.3754x over previous
import functools

import jax
import jax.numpy as jnp
from jax import lax
from jax.experimental import pallas as pl
from jax.experimental.pallas import tpu as pltpu
from jax.experimental.pallas import tpu_sc as plsc

FIELD = 26
BATCH = 16384
LANES = 16
NC = 2
NS = 16
NW = NC * NS
ROWS_W = BATCH // NW
GROUPS = ROWS_W // LANES
FLAT = FIELD * ROWS_W


def _sc_body(ids_hbm, vals_hbm, w_hbm, b_hbm, out_hbm,
             idx_v, vals_v, g_v, acc_v, out_v, b_v, sems):
    c = lax.axis_index("c")
    s = lax.axis_index("s")
    wid = s * NC + c

    pltpu.sync_copy(ids_hbm.at[wid], idx_v)
    pltpu.sync_copy(vals_hbm.at[wid], vals_v)
    pltpu.sync_copy(b_hbm, b_v)
    for j in range(FIELD):
        pltpu.async_copy(
            w_hbm.at[idx_v.at[pl.ds(j * ROWS_W, ROWS_W)]],
            g_v.at[pl.ds(j * ROWS_W, ROWS_W)],
            sems.at[j])

    zero = jnp.zeros((LANES,), jnp.float32)
    for t in range(GROUPS):
        acc_v[pl.ds(t * LANES, LANES)] = zero

    def field(j, carry):
        off = j * ROWS_W
        pltpu.make_async_copy(
            w_hbm.at[idx_v.at[pl.ds(off, ROWS_W)]],
            g_v.at[pl.ds(off, ROWS_W)],
            sems.at[j]).wait()
        for t in range(GROUPS):
            w = g_v[pl.ds(off + t * LANES, LANES)]
            v = vals_v[pl.ds(off + t * LANES, LANES)]
            plsc.addupdate(acc_v.at[pl.ds(t * LANES, LANES)], w * v)
        return carry

    lax.fori_loop(0, FIELD, field, 0)

    bias = b_v[...]
    for t in range(GROUPS):
        z = acc_v[pl.ds(t * LANES, LANES)] + bias
        out_v[pl.ds(t * LANES, LANES)] = 1.0 / (1.0 + jnp.exp(-z))
    pltpu.sync_copy(out_v, out_hbm.at[pl.ds(wid * ROWS_W, ROWS_W)])


_sc_kernel = functools.partial(
    pl.kernel,
    out_type=jax.ShapeDtypeStruct((BATCH,), jnp.float32),
    mesh=plsc.VectorSubcoreMesh(core_axis_name="c", subcore_axis_name="s"),
    compiler_params=pltpu.CompilerParams(needs_layout_passes=False),
    scratch_types=[
        pltpu.VMEM((FLAT,), jnp.int32),
        pltpu.VMEM((FLAT,), jnp.float32),
        pltpu.VMEM((FLAT,), jnp.float32),
        pltpu.VMEM((ROWS_W,), jnp.float32),
        pltpu.VMEM((ROWS_W,), jnp.float32),
        pltpu.VMEM((LANES,), jnp.float32),
        pltpu.SemaphoreType.DMA((FIELD,)),
    ],
)(_sc_body)


def kernel(feat_ids, feat_vals, LR_W, LR_B):
    ids_t = feat_ids.reshape(NW, ROWS_W, FIELD).transpose(0, 2, 1).reshape(NW, FLAT)
    vals_t = feat_vals.reshape(NW, ROWS_W, FIELD).transpose(0, 2, 1).reshape(NW, FLAT)
    b16 = jnp.broadcast_to(LR_B, (LANES,))
    return _sc_kernel(ids_t, vals_t, LR_W, b16)
